# baseline (speedup 1.0000x reference)
import functools

import jax
import jax.numpy as jnp
from jax import lax
from jax.experimental import pallas as pl
from jax.experimental.layout import Format, Layout
from jax.experimental.pallas import tpu as pltpu
from jax.experimental.pallas import tpu_sc as plsc

VOCAB = 1000002
EMBED_DIM = 32
N_TOKENS = 819200

_info = plsc.get_sparse_core_info()
_NW = _info.num_cores * _info.num_subcores
_B_PER_W = N_TOKENS // _NW
_CHUNK = 1280
_N_CHUNKS = _B_PER_W // _CHUNK


def _embed_body(idx_hbm, table_hbm, out_hbm, idx_v, rows_v,
                gsem0, gsem1, ssem0, ssem1):
    wid = lax.axis_index("s") * _info.num_cores + lax.axis_index("c")
    base = wid * _B_PER_W
    gsems = (gsem0, gsem1)
    ssems = (ssem0, ssem1)

    def start_gather(g, b):
        idx_ref = idx_v.at[pl.ds(g * _CHUNK, _CHUNK)]
        pltpu.make_async_copy(table_hbm.at[idx_ref], rows_v.at[b],
                              gsems[b]).start()

    def wait_gather(b):
        pltpu.make_async_copy(table_hbm.at[pl.ds(0, _CHUNK)], rows_v.at[b],
                              gsems[b]).wait()

    def start_store(g, b):
        pltpu.make_async_copy(rows_v.at[b],
                              out_hbm.at[pl.ds(base + g * _CHUNK, _CHUNK)],
                              ssems[b]).start()

    def wait_store(b):
        pltpu.make_async_copy(out_hbm.at[pl.ds(0, _CHUNK)], rows_v.at[b],
                              ssems[b]).wait()

    def step(g, b, first, last):
        wait_gather(b)
        if not last:
            start_gather(g + 1, 1 - b)
        if not first:
            wait_store(b)
        start_store(g, b)

    pltpu.sync_copy(idx_hbm.at[pl.ds(base, _B_PER_W)], idx_v)

    start_gather(0, 0)
    step(0, 0, first=True, last=False)
    step(1, 1, first=True, last=False)
    step(2, 0, first=False, last=False)

    def superstep(ss, carry):
        step(3 + 2 * ss, 1, first=False, last=False)
        step(4 + 2 * ss, 0, first=False, last=False)
        return carry

    lax.fori_loop(0, (_N_CHUNKS - 4) // 2, superstep, 0)

    step(_N_CHUNKS - 1, 1, first=False, last=True)
    wait_store(0)
    wait_store(1)


def _embed(indices, table):
    mesh = plsc.VectorSubcoreMesh(core_axis_name="c", subcore_axis_name="s")
    f = functools.partial(
        pl.kernel,
        mesh=mesh,
        out_type=jax.ShapeDtypeStruct((N_TOKENS, EMBED_DIM), jnp.float32),
        scratch_types=[
            pltpu.VMEM((_B_PER_W,), jnp.int32),
            pltpu.VMEM((2, _CHUNK, EMBED_DIM), jnp.float32),
            pltpu.SemaphoreType.DMA,
            pltpu.SemaphoreType.DMA,
            pltpu.SemaphoreType.DMA,
            pltpu.SemaphoreType.DMA,
        ],
        compiler_params=pltpu.CompilerParams(use_tc_tiling_on_sc=False,
                                             needs_layout_passes=False),
    )(_embed_body)
    return f(indices, table)


@functools.cache
def _jitted():
    dev = jax.devices()[0]
    fmt = Format(Layout((0, 1), ()), jax.sharding.SingleDeviceSharding(dev))
    return jax.jit(_embed, out_shardings=fmt)


def kernel(indices, table):
    return _jitted()(indices, table)


# speedup vs baseline: 1.0005x; 1.0005x over previous
import functools

import jax
import jax.numpy as jnp
from jax import lax
from jax.experimental import pallas as pl
from jax.experimental.layout import Format, Layout
from jax.experimental.pallas import tpu as pltpu
from jax.experimental.pallas import tpu_sc as plsc

VOCAB = 1000002
EMBED_DIM = 32
N_TOKENS = 819200

_info = plsc.get_sparse_core_info()
_NW = _info.num_cores * _info.num_subcores
_B_PER_W = N_TOKENS // _NW
_CHUNK = 1280
_N_CHUNKS = _B_PER_W // _CHUNK


def _embed_body(idx_hbm, table_hbm, out_hbm, idx_v, rows_v,
                gsem0, gsem1, ssem0, ssem1):
    wid = lax.axis_index("s") * _info.num_cores + lax.axis_index("c")
    base = wid * _B_PER_W
    gsems = (gsem0, gsem1)
    ssems = (ssem0, ssem1)

    def start_gather(g, b):
        idx_ref = idx_v.at[pl.ds(g * _CHUNK, _CHUNK)]
        pltpu.make_async_copy(table_hbm.at[idx_ref], rows_v.at[b],
                              gsems[b]).start()

    def wait_gather(b):
        pltpu.make_async_copy(table_hbm.at[pl.ds(0, _CHUNK)], rows_v.at[b],
                              gsems[b]).wait()

    def start_store(g, b):
        pltpu.make_async_copy(rows_v.at[b],
                              out_hbm.at[pl.ds(base + g * _CHUNK, _CHUNK)],
                              ssems[b]).start()

    def wait_store(b):
        pltpu.make_async_copy(out_hbm.at[pl.ds(0, _CHUNK)], rows_v.at[b],
                              ssems[b]).wait()

    def step(g, b, first, last):
        wait_gather(b)
        if not last:
            start_gather(g + 1, 1 - b)
        if not first:
            wait_store(b)
        start_store(g, b)

    pltpu.sync_copy(idx_hbm.at[pl.ds(base, _B_PER_W)], idx_v)

    start_gather(0, 0)
    step(0, 0, first=True, last=False)
    step(1, 1, first=True, last=False)
    step(2, 0, first=False, last=False)

    def superstep(ss, carry):
        step(3 + 2 * ss, 1, first=False, last=False)
        step(4 + 2 * ss, 0, first=False, last=False)
        return carry

    lax.fori_loop(0, (_N_CHUNKS - 4) // 2, superstep, 0)

    step(_N_CHUNKS - 1, 1, first=False, last=True)
    wait_store(0)
    wait_store(1)


def _embed(indices, table):
    mesh = plsc.VectorSubcoreMesh(core_axis_name="c", subcore_axis_name="s")
    f = functools.partial(
        pl.kernel,
        mesh=mesh,
        out_type=jax.ShapeDtypeStruct((N_TOKENS, EMBED_DIM), jnp.float32),
        scratch_types=[
            pltpu.VMEM((_B_PER_W,), jnp.int32),
            pltpu.VMEM((2, _CHUNK, EMBED_DIM), jnp.float32),
            pltpu.SemaphoreType.DMA,
            pltpu.SemaphoreType.DMA,
            pltpu.SemaphoreType.DMA,
            pltpu.SemaphoreType.DMA,
        ],
        compiler_params=pltpu.CompilerParams(use_tc_tiling_on_sc=False,
                                             needs_layout_passes=False),
    )(_embed_body)
    return f(indices, table)


_jit_embed = jax.jit(_embed)


def kernel(indices, table):
    return _jit_embed(indices, table)
